# baseline (speedup 1.0000x reference)
import functools

import jax
import jax.numpy as jnp
from jax import lax
from jax.experimental import pallas as pl
from jax.experimental.pallas import tpu as pltpu
from jax.experimental.pallas import tpu_sc as plsc

B = 16384
D = 64
L = 16
NC = 2
NS = 16
NW = NC * NS
BPW = B // NW
CH = 256
NCHUNK = BPW // CH
GPC = CH // L
KUNROLL = 16
NQ = 8


def _sc_body(users_hbm, items_hbm, ut_hbm, it_hbm, pop_hbm, out_hbm,
             uidx_v, iidx_v, urow_v, irow_v, pop_v, out_v,
             sem_u, sem_i, sem_p):
    wid = lax.axis_index("s") * NC + lax.axis_index("c")

    lanes = lax.iota(jnp.int32, L)

    def chunk_body(ci, _):
        base = wid * BPW + ci * CH
        pltpu.sync_copy(users_hbm.at[pl.ds(base, CH)], uidx_v)
        pltpu.sync_copy(items_hbm.at[pl.ds(base, CH)], iidx_v)

        cp = pltpu.async_copy(pop_hbm.at[iidx_v], pop_v, sem_p)

        def fire_body(j, _):
            uv = uidx_v[pl.ds(j * KUNROLL, KUNROLL)]
            iv = iidx_v[pl.ds(j * KUNROLL, KUNROLL)]
            for kk in range(KUNROLL):
                k = j * KUNROLL + kk
                pltpu.async_copy(ut_hbm.at[uv[kk]], urow_v.at[k],
                                 sem_u.at[kk % NQ])
                pltpu.async_copy(it_hbm.at[iv[kk]], irow_v.at[k],
                                 sem_i.at[kk % NQ])
            return 0

        lax.fori_loop(0, CH // KUNROLL, fire_body, 0)

        def drain_body(j, _):
            for q in range(NQ):
                pltpu.make_async_copy(ut_hbm.at[0], urow_v.at[0],
                                      sem_u.at[q]).wait()
                pltpu.make_async_copy(it_hbm.at[0], irow_v.at[0],
                                      sem_i.at[q]).wait()
            return 0

        lax.fori_loop(0, CH // NQ, drain_body, 0)
        cp.wait()

        def g_body(g, _):
            row = g * L + lanes

            def d_body(d, accs):
                a0, a1 = accs
                c0 = jnp.full((L,), 2 * d, jnp.int32)
                c1 = c0 + 1
                u0 = plsc.load_gather(urow_v, [row, c0])
                i0 = plsc.load_gather(irow_v, [row, c0])
                u1 = plsc.load_gather(urow_v, [row, c1])
                i1 = plsc.load_gather(irow_v, [row, c1])
                return (a0 + u0 * i0, a1 + u1 * i1)

            zero = jnp.zeros((L,), jnp.float32)
            a0, a1 = lax.fori_loop(0, D // 2, d_body, (zero, zero))
            acc = a0 + a1
            r = jnp.where(acc > 0, acc + 1.0, jnp.exp(acc))
            p = pop_v[pl.ds(g * L, L)]
            out_v[pl.ds(g * L, L)] = r * p
            return 0

        lax.fori_loop(0, GPC, g_body, 0)
        pltpu.sync_copy(out_v, out_hbm.at[pl.ds(base, CH)])
        return 0

    lax.fori_loop(0, NCHUNK, chunk_body, 0)


@functools.partial(jax.jit)
def _run(users, items, user_table, item_table, last_popularity):
    mesh = plsc.VectorSubcoreMesh(core_axis_name="c", subcore_axis_name="s")
    f = functools.partial(
        pl.kernel,
        mesh=mesh,
        out_type=jax.ShapeDtypeStruct((B,), jnp.float32),
        scratch_types=[
            pltpu.VMEM((CH,), jnp.int32),
            pltpu.VMEM((CH,), jnp.int32),
            pltpu.VMEM((CH, D), jnp.float32),
            pltpu.VMEM((CH, D), jnp.float32),
            pltpu.VMEM((CH,), jnp.float32),
            pltpu.VMEM((CH,), jnp.float32),
            pltpu.SemaphoreType.DMA((NQ,)),
            pltpu.SemaphoreType.DMA((NQ,)),
            pltpu.SemaphoreType.DMA,
        ],
        compiler_params=pltpu.CompilerParams(
            use_tc_tiling_on_sc=True, needs_layout_passes=False),
    )(_sc_body)
    return f(users, items, user_table, item_table, last_popularity)


def kernel(users, items, user_table, item_table, last_popularity):
    return _run(users.astype(jnp.int32), items.astype(jnp.int32),
                user_table, item_table, last_popularity)


# speedup vs baseline: 1.0743x; 1.0743x over previous
import functools

import jax
import jax.numpy as jnp
from jax import lax
from jax.experimental import pallas as pl
from jax.experimental.pallas import tpu as pltpu
from jax.experimental.pallas import tpu_sc as plsc

B = 16384
D = 64
L = 16
NC = 2
NS = 16
NW = NC * NS
BPW = B // NW
CH = 128
NCHUNK = BPW // CH
GPC = CH // L
KUNROLL = 16
NBUF = 2


def _sc_body(users_hbm, items_hbm, ut_hbm, it_hbm, pop_hbm, out_hbm,
             uidx_v, iidx_v, urow_v, irow_v, pop_v, out_v,
             sem_u, sem_i, sem_p):
    wid = lax.axis_index("s") * NC + lax.axis_index("c")

    lanes = lax.iota(jnp.int32, L)

    def fire(c, slot):
        base = wid * BPW + c * CH
        pltpu.sync_copy(users_hbm.at[pl.ds(base, CH)], uidx_v)
        pltpu.sync_copy(items_hbm.at[pl.ds(base, CH)], iidx_v)
        cp = pltpu.async_copy(pop_hbm.at[iidx_v], pop_v.at[slot],
                              sem_p.at[slot])

        def fire_body(j, _):
            uv = uidx_v[pl.ds(j * KUNROLL, KUNROLL)]
            iv = iidx_v[pl.ds(j * KUNROLL, KUNROLL)]
            for kk in range(KUNROLL):
                k = j * KUNROLL + kk
                pltpu.async_copy(ut_hbm.at[uv[kk]], urow_v.at[slot, k],
                                 sem_u.at[slot])
                pltpu.async_copy(it_hbm.at[iv[kk]], irow_v.at[slot, k],
                                 sem_i.at[slot])
            return 0

        lax.fori_loop(0, CH // KUNROLL, fire_body, 0)
        return cp

    def drain_and_compute(c, slot, cp):
        base = wid * BPW + c * CH

        def drain_body(j, _):
            pltpu.make_async_copy(ut_hbm.at[0], urow_v.at[0, 0],
                                  sem_u.at[slot]).wait()
            pltpu.make_async_copy(it_hbm.at[0], irow_v.at[0, 0],
                                  sem_i.at[slot]).wait()
            return 0

        lax.fori_loop(0, CH, drain_body, 0)
        cp.wait()

        def g_body(g, _):
            row = g * L + lanes

            def d_body(d, accs):
                a0, a1 = accs
                c0 = jnp.full((L,), 2 * d, jnp.int32)
                c1 = c0 + 1
                u0 = plsc.load_gather(urow_v.at[slot], [row, c0])
                i0 = plsc.load_gather(irow_v.at[slot], [row, c0])
                u1 = plsc.load_gather(urow_v.at[slot], [row, c1])
                i1 = plsc.load_gather(irow_v.at[slot], [row, c1])
                return (a0 + u0 * i0, a1 + u1 * i1)

            zero = jnp.zeros((L,), jnp.float32)
            a0, a1 = lax.fori_loop(0, D // 2, d_body, (zero, zero))
            acc = a0 + a1
            r = jnp.where(acc > 0, acc + 1.0, jnp.exp(acc))
            p = pop_v[slot, pl.ds(g * L, L)]
            out_v[pl.ds(g * L, L)] = r * p
            return 0

        lax.fori_loop(0, GPC, g_body, 0)
        pltpu.sync_copy(out_v, out_hbm.at[pl.ds(base, CH)])

    pending = None
    for c in range(NCHUNK + 1):
        if c < NCHUNK:
            cp = fire(c, c % NBUF)
        if c >= 1:
            drain_and_compute(c - 1, (c - 1) % NBUF, pending)
        pending = cp if c < NCHUNK else None


@functools.partial(jax.jit)
def _run(users, items, user_table, item_table, last_popularity):
    mesh = plsc.VectorSubcoreMesh(core_axis_name="c", subcore_axis_name="s")
    f = functools.partial(
        pl.kernel,
        mesh=mesh,
        out_type=jax.ShapeDtypeStruct((B,), jnp.float32),
        scratch_types=[
            pltpu.VMEM((CH,), jnp.int32),
            pltpu.VMEM((CH,), jnp.int32),
            pltpu.VMEM((NBUF, CH, D), jnp.float32),
            pltpu.VMEM((NBUF, CH, D), jnp.float32),
            pltpu.VMEM((NBUF, CH), jnp.float32),
            pltpu.VMEM((CH,), jnp.float32),
            pltpu.SemaphoreType.DMA((NBUF,)),
            pltpu.SemaphoreType.DMA((NBUF,)),
            pltpu.SemaphoreType.DMA((NBUF,)),
        ],
        compiler_params=pltpu.CompilerParams(
            use_tc_tiling_on_sc=True, needs_layout_passes=False),
    )(_sc_body)
    return f(users, items, user_table, item_table, last_popularity)


def kernel(users, items, user_table, item_table, last_popularity):
    return _run(users.astype(jnp.int32), items.astype(jnp.int32),
                user_table, item_table, last_popularity)
